# baseline (speedup 1.0000x reference)
import functools

import jax
import jax.numpy as jnp
from jax import lax
from jax.experimental import pallas as pl
from jax.experimental.pallas import tpu as pltpu
from jax.experimental.pallas import tpu_sc as plsc

D_MODEL = 64
SCALE = 8.0
NUM_CORES = 2
NUM_SUBCORES = 16
NUM_WORKERS = NUM_CORES * NUM_SUBCORES
SUB = 128
K = 8
CHUNK = SUB * K
LANES = 16


@functools.partial(jax.jit, static_argnames=("n",))
def _sc_embed(x2d, table, n):
    per_w = n // NUM_WORKERS
    n_chunks = per_w // CHUNK
    rows_per_chunk = CHUNK // SUB
    mesh = plsc.VectorSubcoreMesh(
        core_axis_name="c",
        subcore_axis_name="s",
        num_cores=NUM_CORES,
        num_subcores=NUM_SUBCORES,
    )

    @functools.partial(
        pl.kernel,
        mesh=mesh,
        out_type=jax.ShapeDtypeStruct((n, D_MODEL), jnp.float32),
        scratch_types=[
            pltpu.VMEM((K, SUB), jnp.int32),
            pltpu.VMEM((CHUNK, D_MODEL), jnp.float32),
            pltpu.SemaphoreType.DMA,
        ],
        compiler_params=pltpu.CompilerParams(use_tc_tiling_on_sc=False),
    )
    def body(x_hbm, tab_hbm, out_hbm, idx_v, rows_v, sem):
        wid = lax.axis_index("s") * NUM_CORES + lax.axis_index("c")
        base = wid * per_w
        base_row = base // SUB

        def chunk_body(g, _):
            off = pl.multiple_of(base + g * CHUNK, CHUNK)
            row0 = pl.multiple_of(base_row + g * rows_per_chunk, rows_per_chunk)
            pltpu.sync_copy(x_hbm.at[pl.ds(row0, rows_per_chunk)], idx_v)
            copies = []
            for j in range(K):
                copies.append(
                    pltpu.async_copy(
                        tab_hbm.at[idx_v.at[j]],
                        rows_v.at[pl.ds(j * SUB, SUB)],
                        sem,
                    )
                )
            for c in copies:
                c.wait()

            def scale_body(r, _):
                for c in range(D_MODEL // LANES):
                    sl = pl.ds(c * LANES, LANES)
                    rows_v[r, sl] = rows_v[r, sl] * SCALE
                return ()

            lax.fori_loop(0, CHUNK, scale_body, (), unroll=2)
            pltpu.sync_copy(rows_v, out_hbm.at[pl.ds(off, CHUNK)])
            return ()

        lax.fori_loop(0, n_chunks, chunk_body, ())

    return body(x2d, table)


def kernel(x, table):
    b, s = x.shape
    n = b * s
    x2d = x.reshape(n // SUB, SUB).astype(jnp.int32)
    out = _sc_embed(x2d, table, n)
    return out.reshape(b, s, D_MODEL)
